# Initial kernel scaffold; baseline (speedup 1.0000x reference)
import functools

import jax
import jax.numpy as jnp
from jax import lax
from jax.experimental import pallas as pl
from jax.experimental.pallas import tpu as pltpu
from jax.experimental.pallas import tpu_sc as plsc

N = 10000
E = 160000
NC = 2
NS = 16
CW = 80
NCH = E // (NS * CW)
ROWS_PER_TILE = N // NS
WB = 125

_MESH = dict(core_axis_name="c", subcore_axis_name="s")


def _prop_body(g_hbm, src_hbm, dst_hbm, u_hbm, src_v, dst_v, rows_v, zbuf, acc, sem):
    c = lax.axis_index("c")
    t = lax.axis_index("s")
    pltpu.sync_copy(src_hbm.at[pl.ds(t * NCH, NCH), :], src_v)
    pltpu.sync_copy(dst_hbm.at[pl.ds(t * NCH, NCH), :], dst_v)

    def adj(k, _):
        i = k // (CW // 16)
        j = (k % (CW // 16)) * 16
        src_v[i, pl.ds(j, 16)] = src_v[i, pl.ds(j, 16)] + c * N
        return 0
    lax.fori_loop(0, NCH * (CW // 16), adj, 0, unroll=False)

    def z16(k, _):
        zbuf[k // 8, pl.ds((k % 8) * 16, 16)] = jnp.zeros((16,), jnp.float32)
        return 0
    lax.fori_loop(0, WB * 8, z16, 0, unroll=False)

    def zacc(i, _):
        pltpu.sync_copy(zbuf, acc.at[pl.ds(t * ROWS_PER_TILE + i * WB, WB), :])
        return 0
    lax.fori_loop(0, ROWS_PER_TILE // WB, zacc, 0, unroll=False)
    plsc.subcore_barrier()

    def chunk(j, _):
        pltpu.async_copy(g_hbm.at[src_v.at[j]], rows_v, sem).wait()
        pltpu.sync_copy(rows_v, acc.at[dst_v.at[j]], add=True)
        return 0
    lax.fori_loop(0, NCH, chunk, 0, unroll=False)
    plsc.subcore_barrier()

    def wb(i, _):
        r = t * ROWS_PER_TILE + i * WB
        pltpu.sync_copy(acc.at[pl.ds(r, WB), :], zbuf)
        pltpu.sync_copy(zbuf, u_hbm.at[pl.ds(c * N + r, WB), :])
        return 0
    lax.fori_loop(0, ROWS_PER_TILE // WB, wb, 0, unroll=False)


_sc_prop = pl.kernel(
    _prop_body,
    out_type=jax.ShapeDtypeStruct((NC * N, 128), jnp.float32),
    mesh=plsc.VectorSubcoreMesh(**_MESH),
    scratch_types=[
        pltpu.VMEM((NCH, CW), jnp.int32),
        pltpu.VMEM((NCH, CW), jnp.int32),
        pltpu.VMEM((CW, 128), jnp.float32),
        pltpu.VMEM((WB, 128), jnp.float32),
        pltpu.VMEM_SHARED((N, 128), jnp.float32),
        pltpu.SemaphoreType.DMA,
    ],
)


def _deg_body(dst_hbm, degp_hbm, dst_v, ones_v, zbuf, acc, sem):
    c = lax.axis_index("c")
    t = lax.axis_index("s")
    pltpu.sync_copy(dst_hbm.at[pl.ds(t * NCH, NCH), :], dst_v)

    def fill(k, _):
        ones_v[k] = jnp.ones((16,), jnp.float32)
        return 0
    lax.fori_loop(0, CW, fill, 0, unroll=False)

    def z16(k, _):
        zbuf[k] = jnp.zeros((16,), jnp.float32)
        return 0
    lax.fori_loop(0, ROWS_PER_TILE, z16, 0, unroll=False)
    pltpu.sync_copy(zbuf, acc.at[pl.ds(t * ROWS_PER_TILE, ROWS_PER_TILE), :])
    plsc.subcore_barrier()

    def chunk(j, _):
        pltpu.sync_copy(ones_v, acc.at[dst_v.at[j]], add=True)
        return 0
    lax.fori_loop(0, NCH, chunk, 0, unroll=False)
    plsc.subcore_barrier()

    pltpu.sync_copy(acc.at[pl.ds(t * ROWS_PER_TILE, ROWS_PER_TILE), :], zbuf)
    pltpu.sync_copy(zbuf, degp_hbm.at[pl.ds(c * N + t * ROWS_PER_TILE, ROWS_PER_TILE), :])


_sc_deg = pl.kernel(
    _deg_body,
    out_type=jax.ShapeDtypeStruct((NC * N, 16), jnp.float32),
    mesh=plsc.VectorSubcoreMesh(**_MESH),
    scratch_types=[
        pltpu.VMEM((NCH, CW), jnp.int32),
        pltpu.VMEM((CW, 16), jnp.float32),
        pltpu.VMEM((ROWS_PER_TILE, 16), jnp.float32),
        pltpu.VMEM_SHARED((N, 16), jnp.float32),
        pltpu.SemaphoreType.DMA,
    ],
)


NB = 1000
NBLK = N // NB


def _prep_kernel(degp_ref, x_ref, dis_ref, dinv_ref, g_ref):
    deg = degp_ref[:, 0]
    dis = jnp.where(deg > 0, lax.rsqrt(jnp.maximum(deg, 1e-12)), 0.0)
    dis_ref[...] = dis
    dinv_ref[...] = dis * dis
    g_ref[...] = x_ref[...] * dis[:, None]


def _tc_prep(degp, x):
    return pl.pallas_call(
        _prep_kernel,
        grid=(NC, NBLK),
        in_specs=[
            pl.BlockSpec((NB, 16), lambda s, i: (i, 0)),
            pl.BlockSpec((NB, 128), lambda s, i: (i, s)),
        ],
        out_specs=[
            pl.BlockSpec((NB,), lambda s, i: (i,)),
            pl.BlockSpec((NB,), lambda s, i: (i,)),
            pl.BlockSpec((NB, 128), lambda s, i: (s * NBLK + i, 0)),
        ],
        out_shape=[
            jax.ShapeDtypeStruct((N,), jnp.float32),
            jax.ShapeDtypeStruct((N,), jnp.float32),
            jax.ShapeDtypeStruct((NC * N, 128), jnp.float32),
        ],
    )(degp, x)


def _rescale_kernel(u_ref, dinv_ref, g_ref):
    g_ref[...] = u_ref[...] * dinv_ref[...][:, None]


def _tc_rescale(u, dinv):
    return pl.pallas_call(
        _rescale_kernel,
        grid=(NC, NBLK),
        in_specs=[
            pl.BlockSpec((NB, 128), lambda s, i: (s * NBLK + i, 0)),
            pl.BlockSpec((NB,), lambda s, i: (i,)),
        ],
        out_specs=pl.BlockSpec((NB, 128), lambda s, i: (s * NBLK + i, 0)),
        out_shape=jax.ShapeDtypeStruct((NC * N, 128), jnp.float32),
    )(u, dinv)


def _combine_kernel(h0_ref, u1l, u1h, u2l, u2h, u3l, u3h, dis_ref, w_ref, b_ref,
                    h_ref, g_ref):
    dis = dis_ref[...]
    f32 = jnp.float32
    acc = jnp.dot(h0_ref[...], w_ref[0], preferred_element_type=f32)
    m = jnp.dot(jnp.concatenate([u1l[...], u1h[...]], axis=1), w_ref[1],
                preferred_element_type=f32)
    m += jnp.dot(jnp.concatenate([u2l[...], u2h[...]], axis=1), w_ref[2],
                 preferred_element_type=f32)
    m += jnp.dot(jnp.concatenate([u3l[...], u3h[...]], axis=1), w_ref[3],
                 preferred_element_type=f32)
    acc = acc + m * dis[:, None] + b_ref[...][None, :]
    h = jnp.maximum(acc, 0.0)
    h_ref[...] = h[:, :]
    g_ref[...] = h[:, :] * dis[:, None]


def _tc_combine(h0, u1, u2, u3, dis, w, b):
    ulo = pl.BlockSpec((NB, 128), lambda s, i: (i, 0))
    uhi = pl.BlockSpec((NB, 128), lambda s, i: (NBLK + i, 0))
    return pl.pallas_call(
        _combine_kernel,
        grid=(NC, NBLK),
        in_specs=[
            pl.BlockSpec((NB, 256), lambda s, i: (i, 0)),
            ulo, uhi, ulo, uhi, ulo, uhi,
            pl.BlockSpec((NB,), lambda s, i: (i,)),
            pl.BlockSpec((4, 256, 128), lambda s, i: (0, 0, s)),
            pl.BlockSpec((128,), lambda s, i: (s,)),
        ],
        out_specs=[
            pl.BlockSpec((NB, 128), lambda s, i: (i, s)),
            pl.BlockSpec((NB, 128), lambda s, i: (s * NBLK + i, 0)),
        ],
        out_shape=[
            jax.ShapeDtypeStruct((N, 256), jnp.float32),
            jax.ShapeDtypeStruct((NC * N, 128), jnp.float32),
        ],
    )(h0, u1, u1, u2, u2, u3, u3, dis, w, b)


def _final_kernel(h0_ref, u1l, u1h, u2l, u2h, u3l, u3h, dis_ref, w_ref, b_ref,
                  out_ref):
    f32 = jnp.float32
    acc = jnp.dot(h0_ref[...], w_ref[0], preferred_element_type=f32)
    m = jnp.dot(jnp.concatenate([u1l[...], u1h[...]], axis=1), w_ref[1],
                preferred_element_type=f32)
    m += jnp.dot(jnp.concatenate([u2l[...], u2h[...]], axis=1), w_ref[2],
                 preferred_element_type=f32)
    m += jnp.dot(jnp.concatenate([u3l[...], u3h[...]], axis=1), w_ref[3],
                 preferred_element_type=f32)
    acc = acc + m * dis_ref[...][:, None] + b_ref[...][None, :]
    out_ref[...] = jnp.tanh(acc)


def _tc_final(h0, u1, u2, u3, dis, w, b):
    ulo = pl.BlockSpec((NB, 128), lambda i: (i, 0))
    uhi = pl.BlockSpec((NB, 128), lambda i: (NBLK + i, 0))
    return pl.pallas_call(
        _final_kernel,
        grid=(NBLK,),
        in_specs=[
            pl.BlockSpec((NB, 256), lambda i: (i, 0)),
            ulo, uhi, ulo, uhi, ulo, uhi,
            pl.BlockSpec((NB,), lambda i: (i,)),
            pl.BlockSpec((4, 256, 64), lambda i: (0, 0, 0)),
            pl.BlockSpec((64,), lambda i: (0,)),
        ],
        out_specs=pl.BlockSpec((NB, 64), lambda i: (i, 0)),
        out_shape=jax.ShapeDtypeStruct((N, 64), jnp.float32),
    )(h0, u1, u1, u2, u2, u3, u3, dis, w, b)


def kernel(x, edge_index, W1, b1, W2, b2, W3, b3):
    src2 = edge_index[0].reshape(NS * NCH, CW)
    dst2 = edge_index[1].reshape(NS * NCH, CW)

    degp = _sc_deg(dst2)
    dis, dinv, g = _tc_prep(degp, x)

    h = x
    for li, (w, b) in enumerate(((W1, b1), (W2, b2), (W3, b3))):
        u1 = _sc_prop(g, src2, dst2)
        g1 = _tc_rescale(u1, dinv)
        u2 = _sc_prop(g1, src2, dst2)
        g2 = _tc_rescale(u2, dinv)
        u3 = _sc_prop(g2, src2, dst2)
        if li < 2:
            h, g = _tc_combine(h, u1, u2, u3, dis, w, b)
        else:
            out = _tc_final(h, u1, u2, u3, dis, w, b)
    return out


# speedup vs baseline: 5.1532x; 5.1532x over previous
import functools

import jax
import jax.numpy as jnp
from jax import lax
from jax.experimental import pallas as pl
from jax.experimental.pallas import tpu as pltpu
from jax.experimental.pallas import tpu_sc as plsc

N = 10000
E = 160000
NC = 2
NS = 16
CW = 80
NCH = E // (NS * CW)


def _mesh():
    return plsc.VectorSubcoreMesh(core_axis_name="c", subcore_axis_name="s",
                                  num_cores=NC, num_subcores=NS)


def _prop_body(g_hbm, src_hbm, dst_hbm, zero_hbm, u_hbm, src_v, dst_v, rows_v,
               acc, sem):
    c = lax.axis_index("c")
    t = lax.axis_index("s")
    @pl.when(t == 0)
    def _():
        pltpu.sync_copy(zero_hbm, acc)

    pltpu.sync_copy(src_hbm.at[t], src_v)
    pltpu.sync_copy(dst_hbm.at[t], dst_v)

    def adj(k, _):
        i = k // (CW // 16)
        j = (k % (CW // 16)) * 16
        src_v[i, pl.ds(j, 16)] = src_v[i, pl.ds(j, 16)] + c * N
        return 0
    lax.fori_loop(0, NCH * (CW // 16), adj, 0, unroll=False)
    plsc.subcore_barrier()

    def chunk(j, _):
        pltpu.async_copy(g_hbm.at[src_v.at[j]], rows_v, sem).wait()
        pltpu.sync_copy(rows_v, acc.at[dst_v.at[j]], add=True)
        return 0
    lax.fori_loop(0, NCH, chunk, 0, unroll=False)
    plsc.subcore_barrier()

    @pl.when(t == 0)
    def _():
        pltpu.sync_copy(acc, u_hbm.at[c])


@functools.lru_cache(maxsize=None)
def _make_sc_prop():
    return pl.kernel(
        _prop_body,
        out_type=jax.ShapeDtypeStruct((NC, N, 128), jnp.float32),
        mesh=_mesh(),
        scratch_types=[
            pltpu.VMEM((NCH, CW), jnp.int32),
            pltpu.VMEM((NCH, CW), jnp.int32),
            pltpu.VMEM((CW, 128), jnp.float32),
            pltpu.VMEM_SHARED((N, 128), jnp.float32),
            pltpu.SemaphoreType.DMA,
        ],
    )


def _sc_prop(g, src3, dst3, zero):
    u = _make_sc_prop()(g, src3, dst3, zero)
    return u.reshape(NC * N, 128)


NB = 1000
NBLK = N // NB


def _prep_kernel(degw_ref, x_ref, disw_ref, dinvw_ref, g_ref):
    deg = degw_ref[...][:, 0:1]
    dis = jnp.where(deg > 0, lax.rsqrt(jnp.maximum(deg, 1e-12)), 0.0)
    disw_ref[...] = jnp.broadcast_to(dis, (NB, 128))
    dinvw_ref[...] = jnp.broadcast_to(dis * dis, (NB, 128))
    g_ref[...] = x_ref[...] * dis


def _tc_prep(degw, x):
    return pl.pallas_call(
        _prep_kernel,
        grid=(NC, NBLK),
        in_specs=[
            pl.BlockSpec((NB, 128), lambda s, i: (i, 0)),
            pl.BlockSpec((NB, 128), lambda s, i: (i, s)),
        ],
        out_specs=[
            pl.BlockSpec((NB, 128), lambda s, i: (i, 0)),
            pl.BlockSpec((NB, 128), lambda s, i: (i, 0)),
            pl.BlockSpec((NB, 128), lambda s, i: (s * NBLK + i, 0)),
        ],
        out_shape=[
            jax.ShapeDtypeStruct((N, 128), jnp.float32),
            jax.ShapeDtypeStruct((N, 128), jnp.float32),
            jax.ShapeDtypeStruct((NC * N, 128), jnp.float32),
        ],
    )(degw, x)


def _rescale_kernel(u_ref, dinvw_ref, g_ref):
    g_ref[...] = u_ref[...] * dinvw_ref[...]


def _tc_rescale(u, dinvw):
    return pl.pallas_call(
        _rescale_kernel,
        grid=(NC, NBLK),
        in_specs=[
            pl.BlockSpec((NB, 128), lambda s, i: (s * NBLK + i, 0)),
            pl.BlockSpec((NB, 128), lambda s, i: (i, 0)),
        ],
        out_specs=pl.BlockSpec((NB, 128), lambda s, i: (s * NBLK + i, 0)),
        out_shape=jax.ShapeDtypeStruct((NC * N, 128), jnp.float32),
    )(u, dinvw)


def _combine_kernel(h0_ref, u1l, u1h, u2l, u2h, u3l, u3h, disw_ref, w_ref, b_ref,
                    h_ref, g_ref):
    dis = disw_ref[...][:, 0:1]
    f32 = jnp.float32
    acc = jnp.dot(h0_ref[...], w_ref[0], preferred_element_type=f32)
    m = jnp.dot(jnp.concatenate([u1l[...], u1h[...]], axis=1), w_ref[1],
                preferred_element_type=f32)
    m += jnp.dot(jnp.concatenate([u2l[...], u2h[...]], axis=1), w_ref[2],
                 preferred_element_type=f32)
    m += jnp.dot(jnp.concatenate([u3l[...], u3h[...]], axis=1), w_ref[3],
                 preferred_element_type=f32)
    acc = acc + m * dis + b_ref[...][None, :]
    h = jnp.maximum(acc, 0.0)
    h_ref[...] = h[:, :]
    g_ref[...] = h[:, :] * dis


def _tc_combine(h0, u1, u2, u3, disw, w, b):
    ulo = pl.BlockSpec((NB, 128), lambda s, i: (i, 0))
    uhi = pl.BlockSpec((NB, 128), lambda s, i: (NBLK + i, 0))
    return pl.pallas_call(
        _combine_kernel,
        grid=(NC, NBLK),
        in_specs=[
            pl.BlockSpec((NB, 256), lambda s, i: (i, 0)),
            ulo, uhi, ulo, uhi, ulo, uhi,
            pl.BlockSpec((NB, 128), lambda s, i: (i, 0)),
            pl.BlockSpec((4, 256, 128), lambda s, i: (0, 0, s)),
            pl.BlockSpec((128,), lambda s, i: (s,)),
        ],
        out_specs=[
            pl.BlockSpec((NB, 128), lambda s, i: (i, s)),
            pl.BlockSpec((NB, 128), lambda s, i: (s * NBLK + i, 0)),
        ],
        out_shape=[
            jax.ShapeDtypeStruct((N, 256), jnp.float32),
            jax.ShapeDtypeStruct((NC * N, 128), jnp.float32),
        ],
    )(h0, u1, u1, u2, u2, u3, u3, disw, w, b)


def _final_kernel(h0_ref, u1l, u1h, u2l, u2h, u3l, u3h, disw_ref, w_ref, b_ref,
                  out_ref):
    f32 = jnp.float32
    acc = jnp.dot(h0_ref[...], w_ref[0], preferred_element_type=f32)
    m = jnp.dot(jnp.concatenate([u1l[...], u1h[...]], axis=1), w_ref[1],
                preferred_element_type=f32)
    m += jnp.dot(jnp.concatenate([u2l[...], u2h[...]], axis=1), w_ref[2],
                 preferred_element_type=f32)
    m += jnp.dot(jnp.concatenate([u3l[...], u3h[...]], axis=1), w_ref[3],
                 preferred_element_type=f32)
    acc = acc + m * disw_ref[...][:, 0:1] + b_ref[...][None, :]
    out_ref[...] = jnp.tanh(acc)


def _tc_final(h0, u1, u2, u3, disw, w, b):
    ulo = pl.BlockSpec((NB, 128), lambda i: (i, 0))
    uhi = pl.BlockSpec((NB, 128), lambda i: (NBLK + i, 0))
    return pl.pallas_call(
        _final_kernel,
        grid=(NBLK,),
        in_specs=[
            pl.BlockSpec((NB, 256), lambda i: (i, 0)),
            ulo, uhi, ulo, uhi, ulo, uhi,
            pl.BlockSpec((NB, 128), lambda i: (i, 0)),
            pl.BlockSpec((4, 256, 64), lambda i: (0, 0, 0)),
            pl.BlockSpec((64,), lambda i: (0,)),
        ],
        out_specs=pl.BlockSpec((NB, 64), lambda i: (i, 0)),
        out_shape=jax.ShapeDtypeStruct((N, 64), jnp.float32),
    )(h0, u1, u1, u2, u2, u3, u3, disw, w, b)


def kernel(x, edge_index, W1, b1, W2, b2, W3, b3):
    src3 = edge_index[0].reshape(NS, NCH, CW)
    dst3 = edge_index[1].reshape(NS, NCH, CW)
    zero = jnp.zeros((N, 128), jnp.float32)
    ones = jnp.ones((NC * N, 128), jnp.float32)

    degw = _sc_prop(ones, src3, dst3, zero)[:N]
    disw, dinvw, g = _tc_prep(degw, x)

    h = x
    for li, (w, b) in enumerate(((W1, b1), (W2, b2), (W3, b3))):
        u1 = _sc_prop(g, src3, dst3, zero)
        g1 = _tc_rescale(u1, dinvw)
        u2 = _sc_prop(g1, src3, dst3, zero)
        g2 = _tc_rescale(u2, dinvw)
        u3 = _sc_prop(g2, src3, dst3, zero)
        if li < 2:
            h, g = _tc_combine(h, u1, u2, u3, disw, w, b)
        else:
            out = _tc_final(h, u1, u2, u3, disw, w, b)
    return out
